# Initial kernel scaffold; baseline (speedup 1.0000x reference)
import functools

import jax
import jax.numpy as jnp
from jax import lax
from jax.experimental import pallas as pl
from jax.experimental.pallas import tpu as pltpu
from jax.experimental.pallas import tpu_sc as plsc

N = 100000
E = 3200000
IN_F = 20
SEG = 6256
NPAD = 16 * SEG
NSUB = E // 128
ROWS = 16
NW = 32
BIG = NSUB // ROWS // NW
EXTRA_BIG = NSUB // ROWS - NW * BIG
TAIL_SUB = NSUB - (NSUB // ROWS) * ROWS
TAIL_BASE = NSUB - TAIL_SUB

_mesh = plsc.VectorSubcoreMesh(core_axis_name="c", subcore_axis_name="s")


@functools.partial(
    pl.kernel,
    mesh=_mesh,
    out_type=jax.ShapeDtypeStruct((2, 3, NPAD), jnp.float32),
    scratch_types=[
        pltpu.VMEM((ROWS, 128), jnp.int32),
        pltpu.VMEM((ROWS, 128), jnp.int32),
        pltpu.VMEM((ROWS, 128), jnp.float32),
        pltpu.VMEM((ROWS, 128), jnp.float32),
        pltpu.VMEM((ROWS, 128), jnp.float32),
        pltpu.VMEM((16,), jnp.int32),
        pltpu.VMEM((16,), jnp.int32),
        pltpu.VMEM((SEG,), jnp.float32),
        pltpu.VMEM_SHARED((NPAD,), jnp.float32),
        pltpu.VMEM_SHARED((NPAD,), jnp.float32),
        pltpu.VMEM_SHARED((NPAD,), jnp.float32),
    ],
)
def _edge_pass(ei3, ew2, hs, av, out, src2d, dst2d, ew2d, vh2d, va2d,
               hs_v, av_v, stage, shd, shh, sha):
    cid = lax.axis_index("c")
    sid = lax.axis_index("s")
    w = cid * 16 + sid

    zero16 = jnp.zeros((16,), jnp.float32)

    def _z(g, carry):
        stage[pl.ds(g * 16, 16)] = zero16
        return carry

    lax.fori_loop(0, SEG // 16, _z, 0)
    seg0 = sid * SEG
    pltpu.sync_copy(stage.at[...], shd.at[pl.ds(seg0, SEG)])
    pltpu.sync_copy(stage.at[...], shh.at[pl.ds(seg0, SEG)])
    pltpu.sync_copy(stage.at[...], sha.at[pl.ds(seg0, SEG)])
    plsc.subcore_barrier()

    pltpu.sync_copy(hs.at[...], hs_v.at[...])
    pltpu.sync_copy(av.at[...], av_v.at[...])
    h16 = hs_v[...]
    a16 = av_v[...]

    def _process(nrows):
        def _row(j, carry):
            def _grp(g, c2):
                sl = pl.ds(g * 16, 16)
                d16 = dst2d[j, sl]
                e16 = ew2d[j, sl]
                vh2d[j, sl] = jnp.where(d16 == h16, e16, 0.0)
                va2d[j, sl] = jnp.where(d16 == a16, e16, 0.0)
                return c2

            lax.fori_loop(0, 8, _grp, 0)
            idx = src2d.at[j]
            pltpu.sync_copy(ew2d.at[j], shd.at[idx], add=True)
            pltpu.sync_copy(vh2d.at[j], shh.at[idx], add=True)
            pltpu.sync_copy(va2d.at[j], sha.at[idx], add=True)
            return carry

        lax.fori_loop(0, nrows, _row, 0)

    nbig = BIG + jnp.where(w < EXTRA_BIG, 1, 0)
    base_big = BIG * w + jnp.minimum(w, EXTRA_BIG)

    def _big(i, carry):
        cb = (base_big + i) * ROWS
        pltpu.sync_copy(ei3.at[0, pl.ds(cb, ROWS), :], src2d.at[...])
        pltpu.sync_copy(ei3.at[1, pl.ds(cb, ROWS), :], dst2d.at[...])
        pltpu.sync_copy(ew2.at[pl.ds(cb, ROWS), :], ew2d.at[...])
        _process(ROWS)
        return carry

    lax.fori_loop(0, nbig, _big, 0)

    @pl.when(w == NW - 1)
    def _tail():
        pltpu.sync_copy(ei3.at[0, pl.ds(TAIL_BASE, TAIL_SUB), :],
                        src2d.at[pl.ds(0, TAIL_SUB)])
        pltpu.sync_copy(ei3.at[1, pl.ds(TAIL_BASE, TAIL_SUB), :],
                        dst2d.at[pl.ds(0, TAIL_SUB)])
        pltpu.sync_copy(ew2.at[pl.ds(TAIL_BASE, TAIL_SUB), :],
                        ew2d.at[pl.ds(0, TAIL_SUB)])
        _process(TAIL_SUB)

    plsc.subcore_barrier()

    pltpu.sync_copy(shd.at[pl.ds(seg0, SEG)], stage.at[...])
    pltpu.sync_copy(stage.at[...], out.at[cid, 0, pl.ds(seg0, SEG)])
    pltpu.sync_copy(shh.at[pl.ds(seg0, SEG)], stage.at[...])
    pltpu.sync_copy(stage.at[...], out.at[cid, 1, pl.ds(seg0, SEG)])
    pltpu.sync_copy(sha.at[pl.ds(seg0, SEG)], stage.at[...])
    pltpu.sync_copy(stage.at[...], out.at[cid, 2, pl.ds(seg0, SEG)])


NB = 12500


def _node_body(tabs_ref, x_ref, out_ref):
    t = tabs_ref[...]
    deg = t[0, 0] + t[1, 0]
    uh = t[0, 1] + t[1, 1]
    ua = t[0, 2] + t[1, 2]
    dis = jnp.where(deg > 0, lax.rsqrt(jnp.where(deg > 0, deg, 1.0)), 0.0)
    stacked = jnp.stack([dis * uh, dis * ua], axis=0)
    part = jnp.dot(stacked, x_ref[...],
                   preferred_element_type=jnp.float32,
                   precision=lax.Precision.HIGHEST)

    @pl.when(pl.program_id(0) == 0)
    def _():
        out_ref[...] = jnp.zeros_like(out_ref)

    out_ref[...] += part


def _node_pass(tabs, x):
    return pl.pallas_call(
        _node_body,
        grid=(N // NB,),
        in_specs=[
            pl.BlockSpec((2, 3, NB), lambda i: (0, 0, i)),
            pl.BlockSpec((NB, IN_F), lambda i: (i, 0)),
        ],
        out_specs=pl.BlockSpec((2, IN_F), lambda i: (0, 0)),
        out_shape=jax.ShapeDtypeStruct((2, IN_F), jnp.float32),
    )(tabs, x)


def kernel(embedding, Wxz, bxz, Whz, bhz, Wxr, bxr, Whr, bhr, Wxh, bxh,
           Whh, bhh, linW, linb, edge_weight, edge_index, home, away):
    home = jnp.asarray(home, jnp.int32)
    away = jnp.asarray(away, jnp.int32)
    ei3 = edge_index.reshape(2, NSUB, 128)
    ew2 = edge_weight.reshape(NSUB, 128)
    hs = jnp.full((16,), home, jnp.int32)
    av = jnp.full((16,), away, jnp.int32)

    tabs = _edge_pass(ei3, ew2, hs, av)
    Y = _node_pass(tabs, embedding)

    def _dis_at(n):
        dg = tabs[0, 0, n] + tabs[1, 0, n]
        return jnp.where(dg > 0, lax.rsqrt(jnp.where(dg > 0, dg, 1.0)), 0.0)

    tx1_h = -_dis_at(home) * Y[0]
    tx1_a = -_dis_at(away) * Y[1]

    def _node_out(xn, tx1):
        z = jax.nn.sigmoid(xn @ Wxz[0] + tx1 @ Wxz[1] + bxz + bhz)
        htil = jnp.tanh(xn @ Wxh[0] + tx1 @ Wxh[1] + bxh + bhh)
        return (1.0 - z) * htil

    feat = jnp.concatenate(
        [_node_out(embedding[away], tx1_a), _node_out(embedding[home], tx1_h)],
        axis=0)
    y = feat @ linW.T + linb
    return jax.nn.softmax(y, axis=0)


# speedup vs baseline: 336.2772x; 336.2772x over previous
import functools

import jax
import jax.numpy as jnp
from jax import lax
from jax.experimental import pallas as pl
from jax.experimental.pallas import tpu as pltpu
from jax.experimental.pallas import tpu_sc as plsc

N = 100000
E = 3200000
IN_F = 20
SEG = 6256
NPAD = 16 * SEG
NSUB = E // 128
ROWS = 16
NW = 32
BIG = NSUB // ROWS // NW
EXTRA_BIG = NSUB // ROWS - NW * BIG
TAIL_SUB = NSUB - (NSUB // ROWS) * ROWS
TAIL_BASE = NSUB - TAIL_SUB

_mesh = plsc.VectorSubcoreMesh(core_axis_name="c", subcore_axis_name="s")


@functools.partial(
    pl.kernel,
    mesh=_mesh,
    out_type=jax.ShapeDtypeStruct((6 * NPAD,), jnp.float32),
    scratch_types=[
        pltpu.VMEM((ROWS, 128), jnp.int32),
        pltpu.VMEM((ROWS, 128), jnp.int32),
        pltpu.VMEM((ROWS, 128), jnp.float32),
        pltpu.VMEM((ROWS, 128), jnp.float32),
        pltpu.VMEM((ROWS, 128), jnp.float32),
        pltpu.VMEM((16,), jnp.int32),
        pltpu.VMEM((16,), jnp.int32),
        pltpu.VMEM((SEG,), jnp.float32),
        pltpu.VMEM_SHARED((NPAD,), jnp.float32),
        pltpu.VMEM_SHARED((NPAD,), jnp.float32),
        pltpu.VMEM_SHARED((NPAD,), jnp.float32),
    ],
)
def _edge_pass(ei3, ew2, hs, av, out, src2d, dst2d, ew2d, vh2d, va2d,
               hs_v, av_v, stage, shd, shh, sha):
    cid = lax.axis_index("c")
    sid = lax.axis_index("s")
    w = cid * 16 + sid

    zero16 = jnp.zeros((16,), jnp.float32)

    def _z(g, carry):
        stage[pl.ds(g * 16, 16)] = zero16
        return carry

    lax.fori_loop(0, SEG // 16, _z, 0)
    seg0 = sid * SEG
    pltpu.sync_copy(stage.at[...], shd.at[pl.ds(seg0, SEG)])
    pltpu.sync_copy(stage.at[...], shh.at[pl.ds(seg0, SEG)])
    pltpu.sync_copy(stage.at[...], sha.at[pl.ds(seg0, SEG)])
    plsc.subcore_barrier()

    pltpu.sync_copy(hs.at[...], hs_v.at[...])
    pltpu.sync_copy(av.at[...], av_v.at[...])
    h16 = hs_v[...]
    a16 = av_v[...]

    def _process(nrows):
        def _row(j, carry):
            def _grp(g, c2):
                sl = pl.ds(g * 16, 16)
                d16 = dst2d[j, sl]
                e16 = ew2d[j, sl]
                vh2d[j, sl] = jnp.where(d16 == h16, e16, 0.0)
                va2d[j, sl] = jnp.where(d16 == a16, e16, 0.0)
                return c2

            lax.fori_loop(0, 8, _grp, 0)
            idx = src2d.at[j]
            pltpu.sync_copy(ew2d.at[j], shd.at[idx], add=True)
            pltpu.sync_copy(vh2d.at[j], shh.at[idx], add=True)
            pltpu.sync_copy(va2d.at[j], sha.at[idx], add=True)
            return carry

        lax.fori_loop(0, nrows, _row, 0)

    nbig = BIG + jnp.where(w < EXTRA_BIG, 1, 0)
    base_big = BIG * w + jnp.minimum(w, EXTRA_BIG)

    def _big(i, carry):
        cb = (base_big + i) * ROWS
        pltpu.sync_copy(ei3.at[0, pl.ds(cb, ROWS), :], src2d.at[...])
        pltpu.sync_copy(ei3.at[1, pl.ds(cb, ROWS), :], dst2d.at[...])
        pltpu.sync_copy(ew2.at[pl.ds(cb, ROWS), :], ew2d.at[...])
        _process(ROWS)
        return carry

    lax.fori_loop(0, nbig, _big, 0)

    @pl.when(w == NW - 1)
    def _tail():
        pltpu.sync_copy(ei3.at[0, pl.ds(TAIL_BASE, TAIL_SUB), :],
                        src2d.at[pl.ds(0, TAIL_SUB)])
        pltpu.sync_copy(ei3.at[1, pl.ds(TAIL_BASE, TAIL_SUB), :],
                        dst2d.at[pl.ds(0, TAIL_SUB)])
        pltpu.sync_copy(ew2.at[pl.ds(TAIL_BASE, TAIL_SUB), :],
                        ew2d.at[pl.ds(0, TAIL_SUB)])
        _process(TAIL_SUB)

    plsc.subcore_barrier()

    obase = cid * (3 * NPAD) + seg0
    pltpu.sync_copy(shd.at[pl.ds(seg0, SEG)], stage.at[...])
    pltpu.sync_copy(stage.at[...], out.at[pl.ds(obase, SEG)])
    pltpu.sync_copy(shh.at[pl.ds(seg0, SEG)], stage.at[...])
    pltpu.sync_copy(stage.at[...], out.at[pl.ds(obase + NPAD, SEG)])
    pltpu.sync_copy(sha.at[pl.ds(seg0, SEG)], stage.at[...])
    pltpu.sync_copy(stage.at[...], out.at[pl.ds(obase + 2 * NPAD, SEG)])


def _node_body(tabs_ref, x_ref, out_ref):
    t = tabs_ref[...]
    deg = t[0, 0] + t[1, 0]
    uh = t[0, 1] + t[1, 1]
    ua = t[0, 2] + t[1, 2]
    dis = jnp.where(deg > 0, lax.rsqrt(jnp.where(deg > 0, deg, 1.0)), 0.0)
    stacked = jnp.stack([dis * uh, dis * ua], axis=0)
    out_ref[...] = jnp.dot(stacked[:, :N], x_ref[...],
                           preferred_element_type=jnp.float32,
                           precision=lax.Precision.HIGHEST)


def _node_pass(tabs, x):
    return pl.pallas_call(
        _node_body,
        out_shape=jax.ShapeDtypeStruct((2, IN_F), jnp.float32),
    )(tabs, x)


def kernel(embedding, Wxz, bxz, Whz, bhz, Wxr, bxr, Whr, bhr, Wxh, bxh,
           Whh, bhh, linW, linb, edge_weight, edge_index, home, away):
    home = jnp.asarray(home, jnp.int32)
    away = jnp.asarray(away, jnp.int32)
    ei3 = edge_index.reshape(2, NSUB, 128)
    ew2 = edge_weight.reshape(NSUB, 128)
    hs = jnp.full((16,), home, jnp.int32)
    av = jnp.full((16,), away, jnp.int32)

    tabs = _edge_pass(ei3, ew2, hs, av).reshape(2, 3, NPAD)
    Y = _node_pass(tabs, embedding)

    def _dis_at(n):
        dg = tabs[0, 0, n] + tabs[1, 0, n]
        return jnp.where(dg > 0, lax.rsqrt(jnp.where(dg > 0, dg, 1.0)), 0.0)

    tx1_h = -_dis_at(home) * Y[0]
    tx1_a = -_dis_at(away) * Y[1]

    def _node_out(xn, tx1):
        z = jax.nn.sigmoid(xn @ Wxz[0] + tx1 @ Wxz[1] + bxz + bhz)
        htil = jnp.tanh(xn @ Wxh[0] + tx1 @ Wxh[1] + bxh + bhh)
        return (1.0 - z) * htil

    feat = jnp.concatenate(
        [_node_out(embedding[away], tx1_a), _node_out(embedding[home], tx1_h)],
        axis=0)
    y = feat @ linW.T + linb
    return jax.nn.softmax(y, axis=0)


# speedup vs baseline: 383.5546x; 1.1406x over previous
import functools

import jax
import jax.numpy as jnp
from jax import lax
from jax.experimental import pallas as pl
from jax.experimental.pallas import tpu as pltpu
from jax.experimental.pallas import tpu_sc as plsc

N = 100000
E = 3200000
IN_F = 20
SEG = 6256
NPAD = 16 * SEG
NSUB = E // 128
ROWS = 16
NW = 32
BIG = NSUB // ROWS // NW
EXTRA_BIG = NSUB // ROWS - NW * BIG
TAIL_SUB = NSUB - (NSUB // ROWS) * ROWS
TAIL_BASE = NSUB - TAIL_SUB

_mesh = plsc.VectorSubcoreMesh(core_axis_name="c", subcore_axis_name="s")


@functools.partial(
    pl.kernel,
    mesh=_mesh,
    out_type=jax.ShapeDtypeStruct((6 * NPAD,), jnp.float32),
    scratch_types=[
        pltpu.VMEM((ROWS, 2, 128), jnp.int32),
        pltpu.VMEM((ROWS * 128,), jnp.float32),
        pltpu.VMEM((ROWS * 128,), jnp.float32),
        pltpu.VMEM((ROWS * 128,), jnp.float32),
        pltpu.VMEM((16,), jnp.int32),
        pltpu.VMEM((16,), jnp.int32),
        pltpu.VMEM((16,), jnp.int32),
        pltpu.VMEM((SEG,), jnp.float32),
        pltpu.VMEM_SHARED((NPAD,), jnp.float32),
        pltpu.VMEM_SHARED((NPAD,), jnp.float32),
        pltpu.VMEM_SHARED((NPAD,), jnp.float32),
        pltpu.SemaphoreType.DMA,
    ],
)
def _edge_pass(ei, ew, hs, av, out, sd, ewb, vhb, vab,
               hs_v, av_v, accm, stage, shd, shh, sha, sem):
    cid = lax.axis_index("c")
    sid = lax.axis_index("s")
    w = cid * 16 + sid

    zero16 = jnp.zeros((16,), jnp.float32)

    def _z(g, carry):
        stage[pl.ds(g * 16, 16)] = zero16
        return carry

    lax.fori_loop(0, SEG // 16, _z, 0)
    seg0 = sid * SEG
    pltpu.sync_copy(stage.at[...], shd.at[pl.ds(seg0, SEG)])
    pltpu.sync_copy(stage.at[...], shh.at[pl.ds(seg0, SEG)])
    pltpu.sync_copy(stage.at[...], sha.at[pl.ds(seg0, SEG)])
    plsc.subcore_barrier()

    pltpu.sync_copy(hs.at[...], hs_v.at[...])
    pltpu.sync_copy(av.at[...], av_v.at[...])
    h16 = hs_v[...]
    a16 = av_v[...]

    def _chunk(cb, nrows):
        cps = [pltpu.async_copy(ei.at[:, pl.ds((cb + j) * 128, 128)],
                                sd.at[j], sem)
               for j in range(nrows)]
        cps.append(pltpu.async_copy(ew.at[pl.ds(cb * 128, nrows * 128)],
                                    ewb.at[pl.ds(0, nrows * 128)], sem))
        for c in cps:
            c.wait()

        def _mrow(j, c):
            def _mgrp(g, c2):
                fl = pl.ds(j * 128 + g * 16, 16)
                d16 = sd[j, 1, pl.ds(g * 16, 16)]
                e16 = ewb[fl]
                vhb[fl] = jnp.where(d16 == h16, e16, 0.0)
                vab[fl] = jnp.where(d16 == a16, e16, 0.0)
                return c2

            lax.fori_loop(0, 8, _mgrp, 0)
            idx = sd.at[j, 0]
            pltpu.sync_copy(ewb.at[pl.ds(j * 128, 128)],
                            shd.at[idx], add=True)
            pltpu.sync_copy(vhb.at[pl.ds(j * 128, 128)],
                            shh.at[idx], add=True)
            pltpu.sync_copy(vab.at[pl.ds(j * 128, 128)],
                            sha.at[idx], add=True)
            return c

        lax.fori_loop(0, nrows, _mrow, 0)

    nbig = BIG + jnp.where(w < EXTRA_BIG, 1, 0)
    base_big = BIG * w + jnp.minimum(w, EXTRA_BIG)

    def _big(i, carry):
        _chunk((base_big + i) * ROWS, ROWS)
        return carry

    lax.fori_loop(0, nbig, _big, 0)

    @pl.when(w == NW - 1)
    def _tail():
        _chunk(TAIL_BASE, TAIL_SUB)

    plsc.subcore_barrier()

    obase = cid * (3 * NPAD) + seg0
    pltpu.sync_copy(shd.at[pl.ds(seg0, SEG)], stage.at[...])
    pltpu.sync_copy(stage.at[...], out.at[pl.ds(obase, SEG)])
    pltpu.sync_copy(shh.at[pl.ds(seg0, SEG)], stage.at[...])
    pltpu.sync_copy(stage.at[...], out.at[pl.ds(obase + NPAD, SEG)])
    pltpu.sync_copy(sha.at[pl.ds(seg0, SEG)], stage.at[...])
    pltpu.sync_copy(stage.at[...], out.at[pl.ds(obase + 2 * NPAD, SEG)])


def _node_body(tabs_ref, x_ref, out_ref):
    t = tabs_ref[...]
    deg = t[0, 0] + t[1, 0]
    uh = t[0, 1] + t[1, 1]
    ua = t[0, 2] + t[1, 2]
    dis = jnp.where(deg > 0, lax.rsqrt(jnp.where(deg > 0, deg, 1.0)), 0.0)
    stacked = jnp.stack([dis * uh, dis * ua], axis=0)
    out_ref[...] = jnp.dot(stacked[:, :N], x_ref[...],
                           preferred_element_type=jnp.float32,
                           precision=lax.Precision.HIGHEST)


def _node_pass(tabs, x):
    return pl.pallas_call(
        _node_body,
        out_shape=jax.ShapeDtypeStruct((2, IN_F), jnp.float32),
    )(tabs, x)


def kernel(embedding, Wxz, bxz, Whz, bhz, Wxr, bxr, Whr, bhr, Wxh, bxh,
           Whh, bhh, linW, linb, edge_weight, edge_index, home, away):
    home = jnp.asarray(home, jnp.int32)
    away = jnp.asarray(away, jnp.int32)
    hs = jnp.full((16,), home, jnp.int32)
    av = jnp.full((16,), away, jnp.int32)

    tabs = _edge_pass(edge_index, edge_weight, hs, av).reshape(2, 3, NPAD)
    Y = _node_pass(tabs, embedding)

    def _dis_at(n):
        dg = tabs[0, 0, n] + tabs[1, 0, n]
        return jnp.where(dg > 0, lax.rsqrt(jnp.where(dg > 0, dg, 1.0)), 0.0)

    tx1_h = -_dis_at(home) * Y[0]
    tx1_a = -_dis_at(away) * Y[1]

    def _node_out(xn, tx1):
        z = jax.nn.sigmoid(xn @ Wxz[0] + tx1 @ Wxz[1] + bxz + bhz)
        htil = jnp.tanh(xn @ Wxh[0] + tx1 @ Wxh[1] + bxh + bhh)
        return (1.0 - z) * htil

    feat = jnp.concatenate(
        [_node_out(embedding[away], tx1_a), _node_out(embedding[home], tx1_h)],
        axis=0)
    y = feat @ linW.T + linb
    return jax.nn.softmax(y, axis=0)


# speedup vs baseline: 605.4060x; 1.5784x over previous
import functools

import jax
import jax.numpy as jnp
from jax import lax
from jax.experimental import pallas as pl
from jax.experimental.pallas import tpu as pltpu
from jax.experimental.pallas import tpu_sc as plsc

N = 100000
E = 3200000
IN_F = 20
SEG = 6256
NPAD = 16 * SEG
NSUB = E // 128
ROWS = 16
NW = 32
BIG = NSUB // ROWS // NW
EXTRA_BIG = NSUB // ROWS - NW * BIG
TAIL_SUB = NSUB - (NSUB // ROWS) * ROWS
TAIL_BASE = NSUB - TAIL_SUB

_mesh = plsc.VectorSubcoreMesh(core_axis_name="c", subcore_axis_name="s")


@functools.partial(
    pl.kernel,
    mesh=_mesh,
    out_type=jax.ShapeDtypeStruct((6 * NPAD,), jnp.float32),
    scratch_types=[
        pltpu.VMEM((ROWS, 2, 128), jnp.int32),
        pltpu.VMEM((ROWS * 128,), jnp.float32),
        pltpu.VMEM((ROWS * 128,), jnp.float32),
        pltpu.VMEM((ROWS * 128,), jnp.float32),
        pltpu.VMEM((16,), jnp.int32),
        pltpu.VMEM((16,), jnp.int32),
        pltpu.VMEM((SEG,), jnp.float32),
        pltpu.VMEM_SHARED((NPAD,), jnp.float32),
        pltpu.VMEM_SHARED((NPAD,), jnp.float32),
        pltpu.VMEM_SHARED((NPAD,), jnp.float32),
        pltpu.SemaphoreType.DMA,
        pltpu.SemaphoreType.DMA,
    ],
)
def _edge_pass(ei, ew, hs, av, out, sd, ewb, vhb, vab,
               hs_v, av_v, stage, shd, shh, sha, sem, ssem):
    cid = lax.axis_index("c")
    sid = lax.axis_index("s")
    w = cid * 16 + sid

    zero16 = jnp.zeros((16,), jnp.float32)

    def _z(g, carry):
        stage[pl.ds(g * 16, 16)] = zero16
        return carry

    lax.fori_loop(0, SEG // 16, _z, 0)
    seg0 = sid * SEG
    pltpu.sync_copy(stage.at[...], shd.at[pl.ds(seg0, SEG)])
    pltpu.sync_copy(stage.at[...], shh.at[pl.ds(seg0, SEG)])
    pltpu.sync_copy(stage.at[...], sha.at[pl.ds(seg0, SEG)])
    plsc.subcore_barrier()

    pltpu.sync_copy(hs.at[...], hs_v.at[...])
    pltpu.sync_copy(av.at[...], av_v.at[...])
    h16 = hs_v[...]
    a16 = av_v[...]

    def _chunk(cb, nrows):
        cps = [pltpu.async_copy(ei.at[:, pl.ds((cb + j) * 128, 128)],
                                sd.at[j], sem)
               for j in range(nrows)]
        cps.append(pltpu.async_copy(ew.at[pl.ds(cb * 128, nrows * 128)],
                                    ewb.at[pl.ds(0, nrows * 128)], sem))
        for c in cps:
            c.wait()

        def _mrow(j, c):
            def _mgrp(g, c2):
                fl = pl.ds(j * 128 + g * 16, 16)
                d16 = sd[j, 1, pl.ds(g * 16, 16)]
                e16 = ewb[fl]
                vhb[fl] = jnp.where(d16 == h16, e16, 0.0)
                vab[fl] = jnp.where(d16 == a16, e16, 0.0)
                return c2

            lax.fori_loop(0, 8, _mgrp, 0)
            return c

        lax.fori_loop(0, nrows, _mrow, 0)

        def _scat(j, c):
            idx = sd.at[j, 0]
            pltpu.async_copy(ewb.at[pl.ds(j * 128, 128)],
                             shd.at[idx], ssem, add=True)
            pltpu.async_copy(vhb.at[pl.ds(j * 128, 128)],
                             shh.at[idx], ssem, add=True)
            pltpu.async_copy(vab.at[pl.ds(j * 128, 128)],
                             sha.at[idx], ssem, add=True)
            return c

        lax.fori_loop(0, nrows, _scat, 0)

        def _drain(j, c):
            idx = sd.at[j, 0]
            pltpu.make_async_copy(ewb.at[pl.ds(j * 128, 128)],
                                  shd.at[idx], ssem).wait()
            pltpu.make_async_copy(vhb.at[pl.ds(j * 128, 128)],
                                  shh.at[idx], ssem).wait()
            pltpu.make_async_copy(vab.at[pl.ds(j * 128, 128)],
                                  sha.at[idx], ssem).wait()
            return c

        lax.fori_loop(0, nrows, _drain, 0)

    nbig = BIG + jnp.where(w < EXTRA_BIG, 1, 0)
    base_big = BIG * w + jnp.minimum(w, EXTRA_BIG)

    def _big(i, carry):
        _chunk((base_big + i) * ROWS, ROWS)
        return carry

    lax.fori_loop(0, nbig, _big, 0)

    @pl.when(w == NW - 1)
    def _tail():
        _chunk(TAIL_BASE, TAIL_SUB)

    plsc.subcore_barrier()

    obase = cid * (3 * NPAD) + seg0
    pltpu.sync_copy(shd.at[pl.ds(seg0, SEG)], stage.at[...])
    pltpu.sync_copy(stage.at[...], out.at[pl.ds(obase, SEG)])
    pltpu.sync_copy(shh.at[pl.ds(seg0, SEG)], stage.at[...])
    pltpu.sync_copy(stage.at[...], out.at[pl.ds(obase + NPAD, SEG)])
    pltpu.sync_copy(sha.at[pl.ds(seg0, SEG)], stage.at[...])
    pltpu.sync_copy(stage.at[...], out.at[pl.ds(obase + 2 * NPAD, SEG)])


def _node_body(tabs_ref, x_ref, out_ref):
    t = tabs_ref[...]
    deg = t[0, 0] + t[1, 0]
    uh = t[0, 1] + t[1, 1]
    ua = t[0, 2] + t[1, 2]
    dis = jnp.where(deg > 0, lax.rsqrt(jnp.where(deg > 0, deg, 1.0)), 0.0)
    stacked = jnp.stack([dis * uh, dis * ua], axis=0)
    out_ref[...] = jnp.dot(stacked[:, :N], x_ref[...],
                           preferred_element_type=jnp.float32,
                           precision=lax.Precision.HIGHEST)


def _node_pass(tabs, x):
    return pl.pallas_call(
        _node_body,
        out_shape=jax.ShapeDtypeStruct((2, IN_F), jnp.float32),
    )(tabs, x)


def kernel(embedding, Wxz, bxz, Whz, bhz, Wxr, bxr, Whr, bhr, Wxh, bxh,
           Whh, bhh, linW, linb, edge_weight, edge_index, home, away):
    home = jnp.asarray(home, jnp.int32)
    away = jnp.asarray(away, jnp.int32)
    hs = jnp.full((16,), home, jnp.int32)
    av = jnp.full((16,), away, jnp.int32)

    tabs = _edge_pass(edge_index, edge_weight, hs, av).reshape(2, 3, NPAD)
    Y = _node_pass(tabs, embedding)

    def _dis_at(n):
        dg = tabs[0, 0, n] + tabs[1, 0, n]
        return jnp.where(dg > 0, lax.rsqrt(jnp.where(dg > 0, dg, 1.0)), 0.0)

    tx1_h = -_dis_at(home) * Y[0]
    tx1_a = -_dis_at(away) * Y[1]

    def _node_out(xn, tx1):
        z = jax.nn.sigmoid(xn @ Wxz[0] + tx1 @ Wxz[1] + bxz + bhz)
        htil = jnp.tanh(xn @ Wxh[0] + tx1 @ Wxh[1] + bxh + bhh)
        return (1.0 - z) * htil

    feat = jnp.concatenate(
        [_node_out(embedding[away], tx1_a), _node_out(embedding[home], tx1_h)],
        axis=0)
    y = feat @ linW.T + linb
    return jax.nn.softmax(y, axis=0)


# speedup vs baseline: 647.4355x; 1.0694x over previous
import functools

import jax
import jax.numpy as jnp
from jax import lax
from jax.experimental import pallas as pl
from jax.experimental.pallas import tpu as pltpu
from jax.experimental.pallas import tpu_sc as plsc

N = 100000
E = 3200000
IN_F = 20
SEG = 6256
NPAD = 16 * SEG
NSUB = E // 128
ROWS = 16
NW = 32
BIG = NSUB // ROWS // NW
EXTRA_BIG = NSUB // ROWS - NW * BIG
TAIL_SUB = NSUB - (NSUB // ROWS) * ROWS
TAIL_BASE = NSUB - TAIL_SUB

_mesh = plsc.VectorSubcoreMesh(core_axis_name="c", subcore_axis_name="s")


@functools.partial(
    pl.kernel,
    mesh=_mesh,
    out_type=jax.ShapeDtypeStruct((6 * NPAD,), jnp.float32),
    scratch_types=[
        pltpu.VMEM((ROWS, 2, 128), jnp.int32),
        pltpu.VMEM((ROWS * 128,), jnp.float32),
        pltpu.VMEM((ROWS * 128,), jnp.float32),
        pltpu.VMEM((ROWS * 128,), jnp.float32),
        pltpu.VMEM((16,), jnp.int32),
        pltpu.VMEM((16,), jnp.int32),
        pltpu.VMEM((SEG,), jnp.float32),
        pltpu.VMEM_SHARED((NPAD,), jnp.float32),
        pltpu.VMEM_SHARED((NPAD,), jnp.float32),
        pltpu.VMEM_SHARED((NPAD,), jnp.float32),
        pltpu.SemaphoreType.DMA,
        pltpu.SemaphoreType.DMA,
    ],
)
def _edge_pass(ei, ew, hs, av, out, sd, ewb, vhb, vab,
               hs_v, av_v, stage, shd, shh, sha, sem, ssem):
    cid = lax.axis_index("c")
    sid = lax.axis_index("s")
    w = cid * 16 + sid

    zero16 = jnp.zeros((16,), jnp.float32)

    def _z(g, carry):
        stage[pl.ds(g * 16, 16)] = zero16
        return carry

    lax.fori_loop(0, SEG // 16, _z, 0)
    seg0 = sid * SEG
    pltpu.sync_copy(stage.at[...], shd.at[pl.ds(seg0, SEG)])
    pltpu.sync_copy(stage.at[...], shh.at[pl.ds(seg0, SEG)])
    pltpu.sync_copy(stage.at[...], sha.at[pl.ds(seg0, SEG)])
    plsc.subcore_barrier()

    pltpu.sync_copy(hs.at[...], hs_v.at[...])
    pltpu.sync_copy(av.at[...], av_v.at[...])
    h16 = hs_v[...]
    a16 = av_v[...]

    def _chunk(cb, nrows):
        cps = [pltpu.async_copy(ei.at[:, pl.ds((cb + j) * 128, 128)],
                                sd.at[j], sem)
               for j in range(nrows)]
        cps.append(pltpu.async_copy(ew.at[pl.ds(cb * 128, nrows * 128)],
                                    ewb.at[pl.ds(0, nrows * 128)], sem))
        for c in cps:
            c.wait()

        def _mrow(j, c):
            for g in range(8):
                fl = pl.ds(j * 128 + g * 16, 16)
                d16 = sd[j, 1, pl.ds(g * 16, 16)]
                e16 = ewb[fl]
                vhb[fl] = jnp.where(d16 == h16, e16, 0.0)
                vab[fl] = jnp.where(d16 == a16, e16, 0.0)
            return c

        lax.fori_loop(0, nrows, _mrow, 0)

        def _scat(j, c):
            idx = sd.at[j, 0]
            pltpu.async_copy(ewb.at[pl.ds(j * 128, 128)],
                             shd.at[idx], ssem, add=True)
            pltpu.async_copy(vhb.at[pl.ds(j * 128, 128)],
                             shh.at[idx], ssem, add=True)
            pltpu.async_copy(vab.at[pl.ds(j * 128, 128)],
                             sha.at[idx], ssem, add=True)
            return c

        lax.fori_loop(0, nrows, _scat, 0)

        def _drain(j, c):
            idx = sd.at[j, 0]
            pltpu.make_async_copy(ewb.at[pl.ds(j * 128, 128)],
                                  shd.at[idx], ssem).wait()
            pltpu.make_async_copy(vhb.at[pl.ds(j * 128, 128)],
                                  shh.at[idx], ssem).wait()
            pltpu.make_async_copy(vab.at[pl.ds(j * 128, 128)],
                                  sha.at[idx], ssem).wait()
            return c

        lax.fori_loop(0, nrows, _drain, 0)

    nbig = BIG + jnp.where(w < EXTRA_BIG, 1, 0)
    base_big = BIG * w + jnp.minimum(w, EXTRA_BIG)

    def _big(i, carry):
        _chunk((base_big + i) * ROWS, ROWS)
        return carry

    lax.fori_loop(0, nbig, _big, 0)

    @pl.when(w == NW - 1)
    def _tail():
        _chunk(TAIL_BASE, TAIL_SUB)

    plsc.subcore_barrier()

    obase = cid * (3 * NPAD) + seg0
    pltpu.sync_copy(shd.at[pl.ds(seg0, SEG)], stage.at[...])
    pltpu.sync_copy(stage.at[...], out.at[pl.ds(obase, SEG)])
    pltpu.sync_copy(shh.at[pl.ds(seg0, SEG)], stage.at[...])
    pltpu.sync_copy(stage.at[...], out.at[pl.ds(obase + NPAD, SEG)])
    pltpu.sync_copy(sha.at[pl.ds(seg0, SEG)], stage.at[...])
    pltpu.sync_copy(stage.at[...], out.at[pl.ds(obase + 2 * NPAD, SEG)])


def _node_body(hm_ref, aw_ref, tabs_ref, x_ref, wxz_ref, bz_ref,
               wxh_ref, bh_ref, linw_ref, linb_ref, out_ref):
    t = tabs_ref[...]
    deg = t[0, 0] + t[1, 0]
    uh = t[0, 1] + t[1, 1]
    ua = t[0, 2] + t[1, 2]
    dis = jnp.where(deg > 0, lax.rsqrt(jnp.where(deg > 0, deg, 1.0)), 0.0)
    stacked = jnp.stack([dis * uh, dis * ua], axis=0)
    Y = jnp.dot(stacked[:, :N], x_ref[...],
                preferred_element_type=jnp.float32,
                precision=lax.Precision.HIGHEST)

    wxz = wxz_ref[...]
    wxh = wxh_ref[...]
    bz = bz_ref[...]
    bh = bh_ref[...]

    def _node_out(node_i, yrow):
        base = pl.multiple_of((node_i // 128) * 128, 128)
        off = node_i - base
        dwin = (tabs_ref[0, 0, pl.ds(base, 128)]
                + tabs_ref[1, 0, pl.ds(base, 128)])
        sel = lax.broadcasted_iota(jnp.int32, (128,), 0) == off
        dg = jnp.sum(jnp.where(sel, dwin, 0.0))
        dn = jnp.where(dg > 0, lax.rsqrt(jnp.where(dg > 0, dg, 1.0)), 0.0)
        base8 = pl.multiple_of((node_i // 8) * 8, 8)
        off8 = node_i - base8
        xw = x_ref[pl.ds(base8, 8), :]
        sel8 = lax.broadcasted_iota(jnp.int32, (8, 1), 0) == off8
        xn = jnp.sum(jnp.where(sel8, xw, 0.0), axis=0, keepdims=True)
        tx1 = -dn * yrow
        z = jax.nn.sigmoid(jnp.dot(xn, wxz[0]) + jnp.dot(tx1, wxz[1]) + bz)
        htil = jnp.tanh(jnp.dot(xn, wxh[0]) + jnp.dot(tx1, wxh[1]) + bh)
        return (1.0 - z) * htil

    h_home = _node_out(hm_ref[0], Y[0:1])
    h_away = _node_out(aw_ref[0], Y[1:2])
    feat = jnp.concatenate([h_away, h_home], axis=1)
    y = lax.dot_general(feat, linw_ref[...],
                        (((1,), (1,)), ((), ())),
                        preferred_element_type=jnp.float32)
    y = y + linb_ref[...][None, :]
    m = jnp.max(y, axis=1, keepdims=True)
    e = jnp.exp(y - m)
    out_ref[...] = e / jnp.sum(e, axis=1, keepdims=True)


def _node_pass(hm, aw, tabs, x, wxz, bz, wxh, bh, linw, linb):
    nsmem = pl.BlockSpec(memory_space=pltpu.SMEM)
    return pl.pallas_call(
        _node_body,
        in_specs=[nsmem, nsmem] + [pl.BlockSpec()] * 8,
        out_shape=jax.ShapeDtypeStruct((1, 3), jnp.float32),
    )(hm, aw, tabs, x, wxz, bz, wxh, bh, linw, linb)


def kernel(embedding, Wxz, bxz, Whz, bhz, Wxr, bxr, Whr, bhr, Wxh, bxh,
           Whh, bhh, linW, linb, edge_weight, edge_index, home, away):
    home = jnp.asarray(home, jnp.int32)
    away = jnp.asarray(away, jnp.int32)
    hs = jnp.full((16,), home, jnp.int32)
    av = jnp.full((16,), away, jnp.int32)

    tabs = _edge_pass(edge_index, edge_weight, hs, av).reshape(2, 3, NPAD)
    out = _node_pass(home.reshape(1), away.reshape(1), tabs, embedding,
                     Wxz, bxz + bhz, Wxh, bxh + bhh, linW, linb)
    return out.reshape(3)


# speedup vs baseline: 689.6387x; 1.0652x over previous
import functools

import jax
import jax.numpy as jnp
from jax import lax
from jax.experimental import pallas as pl
from jax.experimental.pallas import tpu as pltpu
from jax.experimental.pallas import tpu_sc as plsc

N = 100000
E = 3200000
IN_F = 20
SEG = 6256
NPAD = 16 * SEG
NSUB = E // 128
ROWS = 32
NW = 32
BIG = NSUB // ROWS // NW
EXTRA_BIG = NSUB // ROWS - NW * BIG
TAIL_SUB = NSUB - (NSUB // ROWS) * ROWS
TAIL_BASE = NSUB - TAIL_SUB

_mesh = plsc.VectorSubcoreMesh(core_axis_name="c", subcore_axis_name="s")


@functools.partial(
    pl.kernel,
    mesh=_mesh,
    out_type=jax.ShapeDtypeStruct((6 * NPAD,), jnp.float32),
    scratch_types=[
        pltpu.VMEM((ROWS, 2, 128), jnp.int32),
        pltpu.VMEM((ROWS * 128,), jnp.float32),
        pltpu.VMEM((ROWS * 128,), jnp.float32),
        pltpu.VMEM((ROWS * 128,), jnp.float32),
        pltpu.VMEM((16,), jnp.int32),
        pltpu.VMEM((16,), jnp.int32),
        pltpu.VMEM((SEG,), jnp.float32),
        pltpu.VMEM_SHARED((NPAD,), jnp.float32),
        pltpu.VMEM_SHARED((NPAD,), jnp.float32),
        pltpu.VMEM_SHARED((NPAD,), jnp.float32),
        pltpu.SemaphoreType.DMA,
        pltpu.SemaphoreType.DMA,
    ],
)
def _edge_pass(ei, ew, hs, av, out, sd, ewb, vhb, vab,
               hs_v, av_v, stage, shd, shh, sha, sem, ssem):
    cid = lax.axis_index("c")
    sid = lax.axis_index("s")
    w = cid * 16 + sid

    zero16 = jnp.zeros((16,), jnp.float32)

    def _z(g, carry):
        stage[pl.ds(g * 16, 16)] = zero16
        return carry

    lax.fori_loop(0, SEG // 16, _z, 0)
    seg0 = sid * SEG
    pltpu.sync_copy(stage.at[...], shd.at[pl.ds(seg0, SEG)])
    pltpu.sync_copy(stage.at[...], shh.at[pl.ds(seg0, SEG)])
    pltpu.sync_copy(stage.at[...], sha.at[pl.ds(seg0, SEG)])
    plsc.subcore_barrier()

    pltpu.sync_copy(hs.at[...], hs_v.at[...])
    pltpu.sync_copy(av.at[...], av_v.at[...])
    h16 = hs_v[...]
    a16 = av_v[...]

    def _chunk(cb, nrows):
        cps = [pltpu.async_copy(ei.at[:, pl.ds((cb + j) * 128, 128)],
                                sd.at[j], sem)
               for j in range(nrows)]
        cps.append(pltpu.async_copy(ew.at[pl.ds(cb * 128, nrows * 128)],
                                    ewb.at[pl.ds(0, nrows * 128)], sem))
        for c in cps:
            c.wait()

        def _mrow(j, c):
            for g in range(8):
                fl = pl.ds(j * 128 + g * 16, 16)
                d16 = sd[j, 1, pl.ds(g * 16, 16)]
                e16 = ewb[fl]
                vhb[fl] = jnp.where(d16 == h16, e16, 0.0)
                vab[fl] = jnp.where(d16 == a16, e16, 0.0)
            return c

        lax.fori_loop(0, nrows, _mrow, 0)

        def _scat(j, c):
            idx = sd.at[j, 0]
            pltpu.async_copy(ewb.at[pl.ds(j * 128, 128)],
                             shd.at[idx], ssem, add=True)
            pltpu.async_copy(vhb.at[pl.ds(j * 128, 128)],
                             shh.at[idx], ssem, add=True)
            pltpu.async_copy(vab.at[pl.ds(j * 128, 128)],
                             sha.at[idx], ssem, add=True)
            return c

        lax.fori_loop(0, nrows, _scat, 0)

        def _drain(j, c):
            idx = sd.at[j, 0]
            pltpu.make_async_copy(ewb.at[pl.ds(j * 128, 128)],
                                  shd.at[idx], ssem).wait()
            pltpu.make_async_copy(vhb.at[pl.ds(j * 128, 128)],
                                  shh.at[idx], ssem).wait()
            pltpu.make_async_copy(vab.at[pl.ds(j * 128, 128)],
                                  sha.at[idx], ssem).wait()
            return c

        lax.fori_loop(0, nrows, _drain, 0)

    nbig = BIG + jnp.where(w < EXTRA_BIG, 1, 0)
    base_big = BIG * w + jnp.minimum(w, EXTRA_BIG)

    def _big(i, carry):
        _chunk((base_big + i) * ROWS, ROWS)
        return carry

    lax.fori_loop(0, nbig, _big, 0)

    @pl.when(w == NW - 1)
    def _tail():
        _chunk(TAIL_BASE, TAIL_SUB)

    plsc.subcore_barrier()

    obase = cid * (3 * NPAD) + seg0
    pltpu.sync_copy(shd.at[pl.ds(seg0, SEG)], stage.at[...])
    pltpu.sync_copy(stage.at[...], out.at[pl.ds(obase, SEG)])
    pltpu.sync_copy(shh.at[pl.ds(seg0, SEG)], stage.at[...])
    pltpu.sync_copy(stage.at[...], out.at[pl.ds(obase + NPAD, SEG)])
    pltpu.sync_copy(sha.at[pl.ds(seg0, SEG)], stage.at[...])
    pltpu.sync_copy(stage.at[...], out.at[pl.ds(obase + 2 * NPAD, SEG)])


def _node_body(hm_ref, aw_ref, tabs_ref, x_ref, wxz_ref, bz_ref,
               wxh_ref, bh_ref, linw_ref, linb_ref, out_ref):
    t = tabs_ref[...]
    deg = t[0, 0] + t[1, 0]
    uh = t[0, 1] + t[1, 1]
    ua = t[0, 2] + t[1, 2]
    dis = jnp.where(deg > 0, lax.rsqrt(jnp.where(deg > 0, deg, 1.0)), 0.0)
    stacked = jnp.stack([dis * uh, dis * ua], axis=0)
    Y = jnp.dot(stacked[:, :N], x_ref[...],
                preferred_element_type=jnp.float32,
                precision=lax.Precision.HIGHEST)

    wxz = wxz_ref[...]
    wxh = wxh_ref[...]
    bz = bz_ref[...]
    bh = bh_ref[...]

    def _node_out(node_i, yrow):
        base = pl.multiple_of((node_i // 128) * 128, 128)
        off = node_i - base
        dwin = (tabs_ref[0, 0, pl.ds(base, 128)]
                + tabs_ref[1, 0, pl.ds(base, 128)])
        sel = lax.broadcasted_iota(jnp.int32, (128,), 0) == off
        dg = jnp.sum(jnp.where(sel, dwin, 0.0))
        dn = jnp.where(dg > 0, lax.rsqrt(jnp.where(dg > 0, dg, 1.0)), 0.0)
        base8 = pl.multiple_of((node_i // 8) * 8, 8)
        off8 = node_i - base8
        xw = x_ref[pl.ds(base8, 8), :]
        sel8 = lax.broadcasted_iota(jnp.int32, (8, 1), 0) == off8
        xn = jnp.sum(jnp.where(sel8, xw, 0.0), axis=0, keepdims=True)
        tx1 = -dn * yrow
        z = jax.nn.sigmoid(jnp.dot(xn, wxz[0]) + jnp.dot(tx1, wxz[1]) + bz)
        htil = jnp.tanh(jnp.dot(xn, wxh[0]) + jnp.dot(tx1, wxh[1]) + bh)
        return (1.0 - z) * htil

    h_home = _node_out(hm_ref[0], Y[0:1])
    h_away = _node_out(aw_ref[0], Y[1:2])
    feat = jnp.concatenate([h_away, h_home], axis=1)
    y = lax.dot_general(feat, linw_ref[...],
                        (((1,), (1,)), ((), ())),
                        preferred_element_type=jnp.float32)
    y = y + linb_ref[...][None, :]
    m = jnp.max(y, axis=1, keepdims=True)
    e = jnp.exp(y - m)
    out_ref[...] = e / jnp.sum(e, axis=1, keepdims=True)


def _node_pass(hm, aw, tabs, x, wxz, bz, wxh, bh, linw, linb):
    nsmem = pl.BlockSpec(memory_space=pltpu.SMEM)
    return pl.pallas_call(
        _node_body,
        in_specs=[nsmem, nsmem] + [pl.BlockSpec()] * 8,
        out_shape=jax.ShapeDtypeStruct((1, 3), jnp.float32),
    )(hm, aw, tabs, x, wxz, bz, wxh, bh, linw, linb)


def kernel(embedding, Wxz, bxz, Whz, bhz, Wxr, bxr, Whr, bhr, Wxh, bxh,
           Whh, bhh, linW, linb, edge_weight, edge_index, home, away):
    home = jnp.asarray(home, jnp.int32)
    away = jnp.asarray(away, jnp.int32)
    hs = jnp.full((16,), home, jnp.int32)
    av = jnp.full((16,), away, jnp.int32)

    tabs = _edge_pass(edge_index, edge_weight, hs, av).reshape(2, 3, NPAD)
    out = _node_pass(home.reshape(1), away.reshape(1), tabs, embedding,
                     Wxz, bxz + bhz, Wxh, bxh + bhh, linW, linb)
    return out.reshape(3)
